# baseline (speedup 1.0000x reference)
import jax
import jax.numpy as jnp
from jax import lax
from jax.experimental import pallas as pl
from jax.experimental.pallas import tpu as pltpu
from jax.experimental.pallas import tpu_sc as plsc

B = 4096
D = 64
S = 200
LANES = 16
NC, NS = 2, 16
NW = NC * NS
BPW = B // NW
S_BASES = tuple(range(0, S - LANES, LANES)) + (S - LANES,)
NBLK = len(S_BASES)
C0, C1 = 104, 96
D_UNROLL = 4


def _body(input_hbm, shorty_hbm, table_hbm, out_hbm,
          idx_v, in_v, rows0, rows1, out_v, sem0, sem1):
    wid = lax.axis_index("c") * NS + lax.axis_index("s")

    pltpu.sync_copy(shorty_hbm.at[wid], idx_v)
    pltpu.sync_copy(input_hbm.at[wid], in_v)

    def start_gather(i, rows, sem):
        pltpu.async_copy(table_hbm.at[idx_v.at[pl.ds(i * S, C0)]],
                         rows.at[pl.ds(0, C0)], sem)
        pltpu.async_copy(table_hbm.at[idx_v.at[pl.ds(i * S + C0, C1)]],
                         rows.at[pl.ds(C0, C1)], sem)

    def wait_gather(rows, sem):
        pltpu.make_async_copy(table_hbm.at[pl.ds(0, S)], rows, sem).wait()

    s_idx = [jnp.int32(sb) + lax.iota(jnp.int32, LANES) for sb in S_BASES]

    def compute(i, rows):
        zero = jnp.zeros((LANES,), jnp.float32)
        accs = (zero,) * NBLK
        for q in range(D // LANES):
            in_q = in_v[pl.ds(i * D + q * LANES, LANES)]

            def dstep(d2, accs_t, q=q, in_q=in_q):
                dv16 = jnp.full((LANES,), d2, jnp.int32)
                xb = in_q.at[dv16].get(
                    mode=lax.GatherScatterMode.PROMISE_IN_BOUNDS)
                dv = jnp.full((LANES,), q * LANES + d2, jnp.int32)
                return tuple(
                    accs_t[k] + plsc.load_gather(rows, [s_idx[k], dv]) * xb
                    for k in range(NBLK))

            accs = lax.fori_loop(0, LANES, dstep, accs, unroll=D_UNROLL)
        for k in range(NBLK):
            out_v[pl.ds(i * S + S_BASES[k], LANES)] = accs[k]

    start_gather(jnp.int32(0), rows0, sem0)

    def step(it, carry):
        g = it * 2
        start_gather(g + 1, rows1, sem1)
        wait_gather(rows0, sem0)
        compute(g, rows0)

        @pl.when(g + 2 < BPW)
        def _():
            start_gather(g + 2, rows0, sem0)

        wait_gather(rows1, sem1)
        compute(g + 1, rows1)
        return carry

    lax.fori_loop(0, BPW // 2, step, 0)
    pltpu.sync_copy(out_v, out_hbm.at[wid])


def kernel(input, lbl_ft, shorty, attn_w):
    w = jax.nn.softmax(attn_w)
    x = (input * w[0]).reshape(NW, BPW * D)
    idx = shorty.astype(jnp.int32).reshape(NW, BPW * S)
    mesh = plsc.VectorSubcoreMesh(core_axis_name="c", subcore_axis_name="s")
    run = pl.kernel(
        _body,
        out_type=jax.ShapeDtypeStruct((NW, BPW * S), jnp.float32),
        mesh=mesh,
        scratch_types=[
            pltpu.VMEM((BPW * S,), jnp.int32),
            pltpu.VMEM((BPW * D,), jnp.float32),
            pltpu.VMEM((S, D), jnp.float32),
            pltpu.VMEM((S, D), jnp.float32),
            pltpu.VMEM((BPW * S,), jnp.float32),
            pltpu.SemaphoreType.DMA,
            pltpu.SemaphoreType.DMA,
        ],
        compiler_params=pltpu.CompilerParams(use_tc_tiling_on_sc=False,
                                             needs_layout_passes=False),
    )
    return run(x, idx, lbl_ft).reshape(B, S)


# speedup vs baseline: 2.0806x; 2.0806x over previous
import jax
import jax.numpy as jnp
from jax import lax
from jax.experimental import pallas as pl
from jax.experimental.pallas import tpu as pltpu
from jax.experimental.pallas import tpu_sc as plsc

B = 4096
D = 64
S = 200
LANES = 16
NC, NS = 2, 16
NW = NC * NS
BPW = B // NW
NBLK = (S + LANES - 1) // LANES
C0, C1 = 104, 96
NBUF = 4
BITREV = (0, 8, 4, 12, 2, 10, 6, 14, 1, 9, 5, 13, 3, 11, 7, 15)


def _body(input_hbm, shorty_hbm, table_hbm, out_hbm,
          idx_v, in_v, rows0, rows1, rows2, rows3, out_v,
          sem0, sem1, sem2, sem3):
    wid = lax.axis_index("c") * NS + lax.axis_index("s")
    bufs = (rows0, rows1, rows2, rows3)
    sems = (sem0, sem1, sem2, sem3)

    pltpu.sync_copy(shorty_hbm.at[wid], idx_v)
    pltpu.sync_copy(input_hbm.at[wid], in_v)

    def start_gather(i, rows, sem):
        pltpu.async_copy(table_hbm.at[idx_v.at[pl.ds(i * S, C0)]],
                         rows.at[pl.ds(0, C0)], sem)
        pltpu.async_copy(table_hbm.at[idx_v.at[pl.ds(i * S + C0, C1)]],
                         rows.at[pl.ds(C0, C1)], sem)

    def wait_gather(rows, sem):
        pltpu.make_async_copy(table_hbm.at[pl.ds(0, S)], rows, sem).wait()

    iota = lax.iota(jnp.int32, LANES)
    shuf_idx = {g: iota ^ (g // 2) for g in (16, 8, 4, 2)}
    shuf_msk = {g: (iota & (g - 1)) < g // 2 for g in (16, 8, 4, 2)}
    bitrev = (((iota & 1) << 3) | ((iota & 2) << 1) |
              ((iota & 4) >> 1) | ((iota & 8) >> 3))

    def shuffle(v, ix):
        return v.at[ix].get(mode=lax.GatherScatterMode.PROMISE_IN_BOUNDS)

    def compute(i, rows):
        xs = [in_v[pl.ds(i * D + c * LANES, LANES)] for c in range(D // LANES)]

        def blk(kb, carry):
            sb = jnp.minimum(kb * LANES, S - LANES)
            vecs = []
            for j in range(LANES):
                acc = rows[sb + j, pl.ds(0, LANES)] * xs[0]
                for c in range(1, D // LANES):
                    acc = acc + rows[sb + j, pl.ds(c * LANES, LANES)] * xs[c]
                vecs.append(acc)
            for g in (16, 8, 4, 2):
                m, ix = shuf_msk[g], shuf_idx[g]
                vecs = [jnp.where(m, a, b) +
                        jnp.where(m, shuffle(a, ix), shuffle(b, ix))
                        for a, b in zip(vecs[0::2], vecs[1::2])]
            out_v[pl.ds(i * S + sb, LANES)] = shuffle(vecs[0], bitrev)
            return carry

        lax.fori_loop(0, NBLK, blk, 0)

    for p in range(NBUF - 1):
        start_gather(jnp.int32(p), bufs[p], sems[p])

    def step(it, carry):
        g = it * NBUF
        for b in range(NBUF):
            i = g + b

            @pl.when(i + NBUF - 1 < BPW)
            def _():
                start_gather(i + NBUF - 1, bufs[(b + NBUF - 1) % NBUF],
                             sems[(b + NBUF - 1) % NBUF])

            wait_gather(bufs[b], sems[b])
            compute(i, bufs[b])
        return carry

    lax.fori_loop(0, BPW // NBUF, step, 0)
    pltpu.sync_copy(out_v, out_hbm.at[wid])


def kernel(input, lbl_ft, shorty, attn_w):
    w = jax.nn.softmax(attn_w)
    x = (input * w[0]).reshape(NW, BPW * D)
    idx = shorty.astype(jnp.int32).reshape(NW, BPW * S)
    mesh = plsc.VectorSubcoreMesh(core_axis_name="c", subcore_axis_name="s")
    run = pl.kernel(
        _body,
        out_type=jax.ShapeDtypeStruct((NW, BPW * S), jnp.float32),
        mesh=mesh,
        scratch_types=[
            pltpu.VMEM((BPW * S,), jnp.int32),
            pltpu.VMEM((BPW * D,), jnp.float32),
            pltpu.VMEM((S, D), jnp.float32),
            pltpu.VMEM((S, D), jnp.float32),
            pltpu.VMEM((S, D), jnp.float32),
            pltpu.VMEM((S, D), jnp.float32),
            pltpu.VMEM((BPW * S,), jnp.float32),
            pltpu.SemaphoreType.DMA,
            pltpu.SemaphoreType.DMA,
            pltpu.SemaphoreType.DMA,
            pltpu.SemaphoreType.DMA,
        ],
        compiler_params=pltpu.CompilerParams(use_tc_tiling_on_sc=False,
                                             needs_layout_passes=False),
    )
    return run(x, idx, lbl_ft).reshape(B, S)
